# baseline (speedup 1.0000x reference)
import functools

import jax
import jax.numpy as jnp
from jax import lax
from jax.experimental import pallas as pl
from jax.experimental.pallas import tpu as pltpu
from jax.experimental.pallas import tpu_sc as plsc

_SELU_ALPHA_SCALE = 1.0507009873554805 * 1.6732632423543772
_SELU_SCALE = 1.0507009873554805


def _sc_gather_body(tab_hbm, idx_hbm, out_hbm, idx_v, rows_v, sem, *,
                    b_per_w, nc):
    wid = lax.axis_index("s") * nc + lax.axis_index("c")
    base = wid * b_per_w
    pltpu.sync_copy(idx_hbm.at[pl.ds(base, b_per_w)], idx_v)
    pltpu.async_copy(tab_hbm.at[idx_v], rows_v, sem).wait()
    pltpu.sync_copy(rows_v, out_hbm.at[pl.ds(base, b_per_w)])


def _sc_gather(table, idx):
    m = idx.shape[0]
    dp = table.shape[1]
    info = plsc.get_sparse_core_info()
    nc, ns = info.num_cores, info.num_subcores
    b_per_w = m // (nc * ns)
    mesh = plsc.VectorSubcoreMesh(core_axis_name="c", subcore_axis_name="s")
    kern = functools.partial(
        pl.kernel,
        out_type=jax.ShapeDtypeStruct((m, dp), jnp.float32),
        mesh=mesh,
        scratch_types=[
            pltpu.VMEM((b_per_w,), jnp.int32),
            pltpu.VMEM((b_per_w, dp), jnp.float32),
            pltpu.SemaphoreType.DMA,
        ],
    )(functools.partial(_sc_gather_body, b_per_w=b_per_w, nc=nc))
    return kern(table, idx)


def _tc_body(u_ref, x_ref, g0_ref, g1_ref, law_ref, lab_ref, fw_ref, fb_ref,
             m_ref, t_ref, prob_ref, pred_ref, situ_ref):
    u = u_ref[...]
    d = u.shape[1]
    s = jnp.dot(u, law_ref[...], preferred_element_type=jnp.float32) + lab_ref[...]
    f = jnp.dot(u, fw_ref[...], preferred_element_type=jnp.float32) + fb_ref[...]
    f = f - jnp.max(f, axis=-1, keepdims=True)
    ef = jnp.exp(f)
    w = ef / jnp.sum(ef, axis=-1, keepdims=True)
    se = w[:, 0:1] * g0_ref[:, :d] + w[:, 1:2] * g1_ref[:, :d]
    situ_ref[...] = se

    x = x_ref[...]

    def col(i):
        return s[:, i:i + 1]

    c_pos = col(0) + _SELU_SCALE * col(4) + col(2) + col(3) + col(6)
    c_neg = col(0) + _SELU_ALPHA_SCALE * col(4)
    c_xneg = col(2)

    pos = x > 0.0
    ax = jnp.abs(x)
    e = jnp.exp(-ax)
    l = jnp.log1p(e)
    r1 = 1.0 / (1.0 + e)
    sig = jnp.where(pos, r1, e * r1)
    e2 = e * e
    r2 = 1.0 / (1.0 + e2)
    th = jnp.where(pos, 2.0 * r2 - 1.0, 1.0 - 2.0 * r2)
    ss = x / (1.0 + ax)
    hsig = jnp.clip(x * (1.0 / 6.0) + 0.5, 0.0, 1.0)

    pred = (jnp.where(pos, c_pos * x, c_xneg * x + c_neg * (e - 1.0))
            + (col(1) + col(8) * x) * hsig
            + col(6) * l + col(7) * ss + col(9) * th + col(5) * sig)
    pred_ref[...] = pred

    se_t = jnp.dot(se, t_ref[...], preferred_element_type=jnp.float32)
    pn2 = jnp.dot(pred * pred, m_ref[...], preferred_element_type=jnp.float32)
    dot = jnp.dot(pred * se_t, m_ref[...], preferred_element_type=jnp.float32)
    sn2 = jnp.sum(se * se, axis=1, keepdims=True)
    prob_ref[...] = dot / jnp.sqrt(pn2) / jnp.sqrt(sn2)


def kernel(u_embeddings, i_embeddings, situ_target_0, situ_target_1,
           la_W, la_b, fusion_W, fusion_b, situ_table_0, situ_table_1):
    b, n, d = i_embeddings.shape
    na = la_W.shape[1]
    ns = fusion_W.shape[1]
    nd = n * d

    dp = 128
    f0 = situ_table_0.shape[0]
    f0p = -(-f0 // 8) * 8
    tab = jnp.concatenate([
        jnp.pad(situ_table_0.astype(jnp.float32), ((0, f0p - f0), (0, dp - d))),
        jnp.pad(situ_table_1.astype(jnp.float32), ((0, 0), (0, dp - d))),
    ], axis=0)
    idx = jnp.concatenate([situ_target_0.astype(jnp.int32),
                           situ_target_1.astype(jnp.int32) + f0p], axis=0)
    g = _sc_gather(tab, idx)

    bb = 256
    grid = b // bb
    x2 = i_embeddings.reshape(b, nd)
    lab2 = la_b.reshape(1, na)
    fb2 = fusion_b.reshape(1, ns)
    karr = jnp.arange(nd, dtype=jnp.int32)
    m_mat = (karr[:, None] // d == jnp.arange(n, dtype=jnp.int32)[None, :]
             ).astype(jnp.float32)
    t_mat = (jnp.arange(d, dtype=jnp.int32)[:, None] == karr[None, :] % d
             ).astype(jnp.float32)

    prob, pred2, situ = pl.pallas_call(
        _tc_body,
        grid=(grid,),
        in_specs=[
            pl.BlockSpec((bb, d), lambda i: (i, 0)),
            pl.BlockSpec((bb, nd), lambda i: (i, 0)),
            pl.BlockSpec((bb, dp), lambda i: (i, 0)),
            pl.BlockSpec((bb, dp), lambda i, o=grid: (i + o, 0)),
            pl.BlockSpec((d, na), lambda i: (0, 0)),
            pl.BlockSpec((1, na), lambda i: (0, 0)),
            pl.BlockSpec((d, ns), lambda i: (0, 0)),
            pl.BlockSpec((1, ns), lambda i: (0, 0)),
            pl.BlockSpec((nd, n), lambda i: (0, 0)),
            pl.BlockSpec((d, nd), lambda i: (0, 0)),
        ],
        out_specs=[
            pl.BlockSpec((bb, n), lambda i: (i, 0)),
            pl.BlockSpec((bb, nd), lambda i: (i, 0)),
            pl.BlockSpec((bb, d), lambda i: (i, 0)),
        ],
        out_shape=[
            jax.ShapeDtypeStruct((b, n), jnp.float32),
            jax.ShapeDtypeStruct((b, nd), jnp.float32),
            jax.ShapeDtypeStruct((b, d), jnp.float32),
        ],
    )(u_embeddings, x2, g, g, la_W, lab2, fusion_W, fb2, m_mat, t_mat)
    return (prob, pred2.reshape(b, n, d), situ)
